# Initial kernel scaffold; baseline (speedup 1.0000x reference)
import jax
import jax.numpy as jnp
from jax import lax
from jax.experimental import pallas as pl
from jax.experimental.pallas import tpu as pltpu
from jax.experimental.pallas import tpu_sc as plsc

TOTAL = 3276800
VOCAB = 1000000
NC = 2
NS = 16
NW = NC * NS
PER_W = TOTAL // NW
CHUNK = 25600
NCHUNK = PER_W // CHUNK
VPAD = 1000064
SEG = VPAD // NS


def _vocab_gather(hashes, table, out, table_sh, idx_v, rows_v, sem):
    cid = lax.axis_index("c")
    sid = lax.axis_index("s")
    wid = sid * NC + cid
    pltpu.sync_copy(table.at[pl.ds(sid * SEG, SEG)],
                    table_sh.at[pl.ds(sid * SEG, SEG)])
    plsc.subcore_barrier()
    base = wid * PER_W
    for i in range(NCHUNK):
        off = base + i * CHUNK
        pltpu.sync_copy(hashes.at[pl.ds(off, CHUNK)], idx_v)
        pltpu.async_copy(table_sh.at[idx_v], rows_v, sem).wait()
        pltpu.sync_copy(rows_v, out.at[pl.ds(off, CHUNK)])


def kernel(token_hashes, start_ids, end_ids, vocab_table):
    table_p = jnp.pad(vocab_table, (0, VPAD - VOCAB))
    mesh = plsc.VectorSubcoreMesh(core_axis_name="c", subcore_axis_name="s")
    gather = pl.kernel(
        _vocab_gather,
        out_type=jax.ShapeDtypeStruct((TOTAL,), jnp.float32),
        mesh=mesh,
        scratch_types=[
            pltpu.VMEM_SHARED((VPAD,), jnp.float32),
            pltpu.VMEM((CHUNK,), jnp.int32),
            pltpu.VMEM((CHUNK,), jnp.float32),
            pltpu.SemaphoreType.DMA,
        ],
    )
    token_ids = gather(token_hashes, table_p)
    return (token_ids, start_ids, end_ids)


# speedup vs baseline: 359.4852x; 359.4852x over previous
import jax
import jax.numpy as jnp
from jax import lax
from jax.experimental import pallas as pl
from jax.experimental.pallas import tpu as pltpu
from jax.experimental.pallas import tpu_sc as plsc

TOTAL = 3276800
VOCAB = 1000000
NC = 2
NS = 16
NW = NC * NS
PER_W = TOTAL // NW
CHUNK = 25600
NCHUNK = PER_W // CHUNK
VPAD = 1000064
SEG = VPAD // NS
SEG_PIECES = (CHUNK, CHUNK, SEG - 2 * CHUNK)


def _vocab_gather(hashes, table, out, table_sh, idx_v, rows_v, sem):
    cid = lax.axis_index("c")
    sid = lax.axis_index("s")
    wid = sid * NC + cid
    for k, sz in enumerate(SEG_PIECES):
        soff = sid * SEG + k * CHUNK
        pltpu.sync_copy(table.at[pl.ds(soff, sz)], rows_v.at[pl.ds(0, sz)])
        pltpu.sync_copy(rows_v.at[pl.ds(0, sz)], table_sh.at[pl.ds(soff, sz)])
    plsc.subcore_barrier()
    base = wid * PER_W
    for i in range(NCHUNK):
        off = base + i * CHUNK
        pltpu.sync_copy(hashes.at[pl.ds(off, CHUNK)], idx_v)
        pltpu.async_copy(table_sh.at[idx_v], rows_v, sem).wait()
        pltpu.sync_copy(rows_v, out.at[pl.ds(off, CHUNK)])


def kernel(token_hashes, start_ids, end_ids, vocab_table):
    table_p = jnp.pad(vocab_table, (0, VPAD - VOCAB))
    mesh = plsc.VectorSubcoreMesh(core_axis_name="c", subcore_axis_name="s")
    gather = pl.kernel(
        _vocab_gather,
        out_type=jax.ShapeDtypeStruct((TOTAL,), jnp.float32),
        mesh=mesh,
        scratch_types=[
            pltpu.VMEM_SHARED((VPAD,), jnp.float32),
            pltpu.VMEM((CHUNK,), jnp.int32),
            pltpu.VMEM((CHUNK,), jnp.float32),
            pltpu.SemaphoreType.DMA,
        ],
    )
    token_ids = gather(token_hashes, table_p)
    return (token_ids, start_ids, end_ids)
